# Initial kernel scaffold; baseline (speedup 1.0000x reference)
import jax
import jax.numpy as jnp
from jax.experimental import pallas as pl


def kernel(x, edge_index, edge_attr, W1, b1, g1, be1, W2, b2, g2, be2, Wl1, bl1, Wl2, bl2):
    raise NotImplementedError("write your pallas kernel here")


# speedup vs baseline: 8.6985x; 8.6985x over previous
import functools

import jax
import jax.numpy as jnp
from jax import lax
from jax.experimental import pallas as pl
from jax.experimental.pallas import tpu as pltpu
from jax.experimental.pallas import tpu_sc as plsc

N = 10000
E = 320000
C_IN = 128
C_HID = 128
C_OUT = 16

NC = 2
NS = 16
NW = NC * NS
EPT = E // NW
K = 80
NCHUNK = EPT // K
RPT = 624
TAIL = N - NS * RPT
ZR = 104

_mesh = plsc.VectorSubcoreMesh(core_axis_name="c", subcore_axis_name="s")


@functools.partial(
    pl.kernel,
    out_type=jax.ShapeDtypeStruct((NC * N,), jnp.float32),
    mesh=_mesh,
    scratch_types=[
        pltpu.VMEM((K,), jnp.int32),
        pltpu.VMEM((K,), jnp.float32),
        pltpu.VMEM((RPT,), jnp.float32),
        pltpu.VMEM_SHARED((N,), jnp.float32),
    ],
)
def _deg_kernel(dst_hbm, ew_hbm, out_hbm, idx_v, ew_v, zbuf, acc_sh):
    c = lax.axis_index("c")
    s = lax.axis_index("s")
    wid = s * NC + c

    def zfill(i, carry):
        zbuf[pl.ds(i * 16, 16)] = jnp.zeros((16,), jnp.float32)
        return carry

    lax.fori_loop(0, RPT // 16, zfill, 0)
    pltpu.sync_copy(zbuf, acc_sh.at[pl.ds(s * RPT, RPT)])

    @pl.when(s == NS - 1)
    def _():
        pltpu.sync_copy(zbuf.at[pl.ds(0, TAIL)],
                        acc_sh.at[pl.ds(NS * RPT, TAIL)])

    plsc.subcore_barrier()
    base = wid * EPT

    def body(i, carry):
        off = base + i * K
        pltpu.sync_copy(dst_hbm.at[pl.ds(off, K)], idx_v)
        pltpu.sync_copy(ew_hbm.at[pl.ds(off, K)], ew_v)
        pltpu.sync_copy(ew_v, acc_sh.at[idx_v], add=True)
        return carry

    lax.fori_loop(0, NCHUNK, body, 0)
    plsc.subcore_barrier()
    pltpu.sync_copy(acc_sh.at[pl.ds(s * RPT, RPT)], zbuf)
    pltpu.sync_copy(zbuf, out_hbm.at[pl.ds(c * N + s * RPT, RPT)])

    @pl.when(s == NS - 1)
    def _():
        pltpu.sync_copy(acc_sh.at[pl.ds(NS * RPT, TAIL)],
                        zbuf.at[pl.ds(0, TAIL)])
        pltpu.sync_copy(zbuf.at[pl.ds(0, TAIL)],
                        out_hbm.at[pl.ds(c * N + NS * RPT, TAIL)])


@functools.partial(
    pl.kernel,
    out_type=jax.ShapeDtypeStruct((NC, N, C_HID), jnp.float32),
    mesh=_mesh,
    scratch_types=[
        pltpu.VMEM((K,), jnp.int32),
        pltpu.VMEM((K,), jnp.int32),
        pltpu.VMEM((K + 16,), jnp.float32),
        pltpu.VMEM((K, C_HID), jnp.float32),
        pltpu.VMEM((ZR, C_HID), jnp.float32),
        pltpu.VMEM_SHARED((N, C_HID), jnp.float32),
        pltpu.SemaphoreType.DMA,
    ],
)
def _scatter_kernel(hsh_hbm, src_hbm, dst_hbm, ew_hbm, out_hbm,
                    sidx, didx, ewv, rows, zbuf, acc_sh, sem):
    c = lax.axis_index("c")
    s = lax.axis_index("s")
    wid = s * NC + c

    def zfill(i, carry):
        r = i // (C_HID // 16)
        col = (i % (C_HID // 16)) * 16
        zbuf[r, pl.ds(col, 16)] = jnp.zeros((16,), jnp.float32)
        return carry

    lax.fori_loop(0, ZR * (C_HID // 16), zfill, 0)
    for r in range(RPT // ZR):
        pltpu.sync_copy(zbuf, acc_sh.at[pl.ds(s * RPT + r * ZR, ZR)])

    @pl.when(s == NS - 1)
    def _():
        pltpu.sync_copy(zbuf.at[pl.ds(0, TAIL)],
                        acc_sh.at[pl.ds(NS * RPT, TAIL)])

    plsc.subcore_barrier()
    base = wid * EPT

    def body(i, carry):
        off = base + i * K
        pltpu.sync_copy(src_hbm.at[pl.ds(off, K)], sidx)
        pltpu.sync_copy(dst_hbm.at[pl.ds(off, K)], didx)
        pltpu.sync_copy(ew_hbm.at[pl.ds(off, K)], ewv.at[pl.ds(0, K)])
        pltpu.async_copy(hsh_hbm.at[sidx], rows, sem).wait()

        def scale(k, carry2):
            v = ewv[pl.ds(k, 16)]
            b = jnp.zeros((16,), jnp.float32) + v[0]
            for j in range(C_HID // 16):
                rows[k, pl.ds(j * 16, 16)] = rows[k, pl.ds(j * 16, 16)] * b
            return carry2

        lax.fori_loop(0, K, scale, 0)
        pltpu.sync_copy(rows, acc_sh.at[didx], add=True)
        return carry

    lax.fori_loop(0, NCHUNK, body, 0)
    plsc.subcore_barrier()
    for r in range(RPT // ZR):
        pltpu.sync_copy(acc_sh.at[pl.ds(s * RPT + r * ZR, ZR)], zbuf)
        pltpu.sync_copy(zbuf, out_hbm.at[c, pl.ds(s * RPT + r * ZR, ZR), :])

    @pl.when(s == NS - 1)
    def _():
        pltpu.sync_copy(acc_sh.at[pl.ds(NS * RPT, TAIL)],
                        zbuf.at[pl.ds(0, TAIL)])
        pltpu.sync_copy(zbuf.at[pl.ds(0, TAIL)],
                        out_hbm.at[c, pl.ds(NS * RPT, TAIL), :])


def _dot(a, b):
    return lax.dot_general(a, b, (((1,), (0,)), ((), ())),
                           precision=lax.Precision.HIGHEST,
                           preferred_element_type=jnp.float32)


def _leaky(x):
    return jnp.where(x > 0, x, 0.01 * x)


def _bn(z, g, b):
    mu = jnp.mean(z, axis=0)
    var = jnp.mean((z - mu) ** 2, axis=0)
    return (z - mu) * lax.rsqrt(var + 1e-5) * g + b


def _tc1_body(x_ref, w1_ref, degp_ref, dinv_ref, hsh_ref):
    deg = degp_ref[pl.ds(0, N)] + degp_ref[pl.ds(N, N)] + 1.0
    dinv = jnp.where(deg > 0, lax.rsqrt(deg), 0.0)
    dinv_ref[...] = dinv
    h = _dot(x_ref[...], w1_ref[...])
    hsh_ref[...] = h * dinv[:, None]


_tc1 = pl.pallas_call(
    _tc1_body,
    out_shape=(jax.ShapeDtypeStruct((N,), jnp.float32),
               jax.ShapeDtypeStruct((N, C_HID), jnp.float32)),
)


def _tc2_body(s_ref, hsh_in_ref, dinv_ref, b1_ref, g1_ref, be1_ref, w2_ref,
              hsh_ref, h_ref):
    dinv = dinv_ref[...]
    z = (s_ref[0] + s_ref[1] + hsh_in_ref[...]) * dinv[:, None] + b1_ref[...]
    h = _leaky(_bn(z, g1_ref[...], be1_ref[...]))
    h_ref[...] = h
    h2 = _dot(h, w2_ref[...])
    hsh_ref[...] = h2 * dinv[:, None]


_tc2 = pl.pallas_call(
    _tc2_body,
    out_shape=(jax.ShapeDtypeStruct((N, C_HID), jnp.float32),
               jax.ShapeDtypeStruct((N, C_HID), jnp.float32)),
)


def _tc3_body(s_ref, hsh_in_ref, dinv_ref, b2_ref, g2_ref, be2_ref, h_ref,
              wl1_ref, bl1_ref, wl2_ref, bl2_ref, out_ref):
    dinv = dinv_ref[...]
    z = (s_ref[0] + s_ref[1] + hsh_in_ref[...]) * dinv[:, None] + b2_ref[...]
    t = _leaky(_bn(z, g2_ref[...], be2_ref[...]) + h_ref[...])
    t = _leaky(_dot(t, wl1_ref[...]) + bl1_ref[...])
    out_ref[...] = _dot(t, wl2_ref[...]) + bl2_ref[...]


_tc3 = pl.pallas_call(
    _tc3_body,
    out_shape=jax.ShapeDtypeStruct((N, C_OUT), jnp.float32),
)


def kernel(x, edge_index, edge_attr, W1, b1, g1, be1, W2, b2, g2, be2,
           Wl1, bl1, Wl2, bl2):
    src = edge_index[0]
    dst = edge_index[1]

    degp = _deg_kernel(dst, edge_attr)
    dinv, hsh1 = _tc1(x, W1, degp)
    s1 = _scatter_kernel(hsh1, src, dst, edge_attr)
    hsh2, h1 = _tc2(s1, hsh1, dinv, b1, g1, be1, W2)
    s2 = _scatter_kernel(hsh2, src, dst, edge_attr)
    return _tc3(s2, hsh2, dinv, b2, g2, be2, h1, Wl1, bl1, Wl2, bl2)


# speedup vs baseline: 17.9937x; 2.0686x over previous
import functools

import jax
import jax.numpy as jnp
from jax import lax
from jax.experimental import pallas as pl
from jax.experimental.pallas import tpu as pltpu
from jax.experimental.pallas import tpu_sc as plsc

N = 10000
E = 320000
C_IN = 128
C_HID = 128
C_OUT = 16

NC = 2
NS = 16
NW = NC * NS
EPT = E // NW
KD = 80
NCHD = EPT // KD
K = 40
NCHUNK = EPT // K
NBUF = 4
RPT = 624
TAIL = N - NS * RPT
QF = RPT // K
QT = RPT - QF * K

_mesh = plsc.VectorSubcoreMesh(core_axis_name="c", subcore_axis_name="s")


@functools.partial(
    pl.kernel,
    out_type=jax.ShapeDtypeStruct((NC * N,), jnp.float32),
    mesh=_mesh,
    scratch_types=[
        pltpu.VMEM((KD,), jnp.int32),
        pltpu.VMEM((KD,), jnp.float32),
        pltpu.VMEM((RPT,), jnp.float32),
        pltpu.VMEM_SHARED((N,), jnp.float32),
    ],
)
def _deg_kernel(dst_hbm, ew_hbm, out_hbm, idx_v, ew_v, zbuf, acc_sh):
    c = lax.axis_index("c")
    s = lax.axis_index("s")
    wid = s * NC + c

    def zfill(i, carry):
        zbuf[pl.ds(i * 16, 16)] = jnp.zeros((16,), jnp.float32)
        return carry

    lax.fori_loop(0, RPT // 16, zfill, 0)
    pltpu.sync_copy(zbuf, acc_sh.at[pl.ds(s * RPT, RPT)])

    @pl.when(s == NS - 1)
    def _():
        pltpu.sync_copy(zbuf.at[pl.ds(0, TAIL)],
                        acc_sh.at[pl.ds(NS * RPT, TAIL)])

    plsc.subcore_barrier()
    base = wid * EPT

    def body(i, carry):
        off = base + i * KD
        pltpu.sync_copy(dst_hbm.at[pl.ds(off, KD)], idx_v)
        pltpu.sync_copy(ew_hbm.at[pl.ds(off, KD)], ew_v)
        pltpu.sync_copy(ew_v, acc_sh.at[idx_v], add=True)
        return carry

    lax.fori_loop(0, NCHD, body, 0)
    plsc.subcore_barrier()
    pltpu.sync_copy(acc_sh.at[pl.ds(s * RPT, RPT)], zbuf)
    pltpu.sync_copy(zbuf, out_hbm.at[pl.ds(c * N + s * RPT, RPT)])

    @pl.when(s == NS - 1)
    def _():
        pltpu.sync_copy(acc_sh.at[pl.ds(NS * RPT, TAIL)],
                        zbuf.at[pl.ds(0, TAIL)])
        pltpu.sync_copy(zbuf.at[pl.ds(0, TAIL)],
                        out_hbm.at[pl.ds(c * N + NS * RPT, TAIL)])


@functools.partial(
    pl.kernel,
    out_type=jax.ShapeDtypeStruct((NC, N, C_HID), jnp.float32),
    mesh=_mesh,
    scratch_types=[
        pltpu.VMEM((NBUF, K), jnp.int32),
        pltpu.VMEM((NBUF, K), jnp.int32),
        pltpu.VMEM((NBUF, K + 16), jnp.float32),
        pltpu.VMEM((NBUF, K, C_HID), jnp.float32),
        pltpu.VMEM_SHARED((N, C_HID), jnp.float32),
        pltpu.SemaphoreType.DMA((NBUF,)),
        pltpu.SemaphoreType.DMA((NBUF,)),
        pltpu.SemaphoreType.DMA((NBUF,)),
    ],
)
def _scatter_kernel(hsh_hbm, src_hbm, dst_hbm, ew_hbm, out_hbm,
                    sbuf, dbuf, ebuf, rows, acc_sh, gsem, ssem, isem):
    c = lax.axis_index("c")
    s = lax.axis_index("s")
    wid = s * NC + c
    base = wid * EPT

    def zfill(i, carry):
        rr = i // (C_HID // 16)
        col = (i % (C_HID // 16)) * 16
        rows[0, rr, pl.ds(col, 16)] = jnp.zeros((16,), jnp.float32)
        return carry

    lax.fori_loop(0, K * (C_HID // 16), zfill, 0)
    for q in range(QF):
        pltpu.sync_copy(rows.at[0], acc_sh.at[pl.ds(s * RPT + q * K, K)])
    pltpu.sync_copy(rows.at[0, pl.ds(0, QT)],
                    acc_sh.at[pl.ds(s * RPT + QF * K, QT)])

    @pl.when(s == NS - 1)
    def _():
        pltpu.sync_copy(rows.at[0, pl.ds(0, TAIL)],
                        acc_sh.at[pl.ds(NS * RPT, TAIL)])

    plsc.subcore_barrier()

    def issue_idx(j, r):
        off = base + j * K
        pltpu.async_copy(src_hbm.at[pl.ds(off, K)], sbuf.at[r], isem.at[r])
        pltpu.async_copy(dst_hbm.at[pl.ds(off, K)], dbuf.at[r], isem.at[r])
        pltpu.async_copy(ew_hbm.at[pl.ds(off, K)],
                         ebuf.at[r, pl.ds(0, K)], isem.at[r])

    def drain_idx(j, r):
        off = base + j * K
        pltpu.make_async_copy(src_hbm.at[pl.ds(off, K)], sbuf.at[r],
                              isem.at[r]).wait()
        pltpu.make_async_copy(dst_hbm.at[pl.ds(off, K)], dbuf.at[r],
                              isem.at[r]).wait()
        pltpu.make_async_copy(ew_hbm.at[pl.ds(off, K)],
                              ebuf.at[r, pl.ds(0, K)], isem.at[r]).wait()

    def issue_gather(r):
        pltpu.async_copy(hsh_hbm.at[sbuf.at[r]], rows.at[r], gsem.at[r])

    issue_idx(0, 0)
    issue_idx(1, 1)
    drain_idx(0, 0)
    issue_gather(0)
    drain_idx(1, 1)
    issue_gather(1)

    NOUT = (NCHUNK + NBUF - 1) // NBUF + 1

    def outer(g, carry):
        for b in range(NBUF):
            i = g * NBUF + b
            rn = (b + 2) % NBUF

            @pl.when(jnp.logical_and(i >= 2, i - 2 < NCHUNK))
            def _():
                pltpu.make_async_copy(rows.at[rn], acc_sh.at[dbuf.at[rn]],
                                      ssem.at[rn]).wait()

            @pl.when(i + 2 < NCHUNK)
            def _():
                issue_idx(i + 2, rn)

            @pl.when(i < NCHUNK)
            def _():
                pltpu.make_async_copy(hsh_hbm.at[sbuf.at[b]], rows.at[b],
                                      gsem.at[b]).wait()

                def scale(k, carry2):
                    v = ebuf[b, pl.ds(k, 16)]
                    bb = jnp.zeros((16,), jnp.float32) + v[0]
                    for jj in range(C_HID // 16):
                        rows[b, k, pl.ds(jj * 16, 16)] = (
                            rows[b, k, pl.ds(jj * 16, 16)] * bb)
                    return carry2

                lax.fori_loop(0, K, scale, 0, unroll=4)
                pltpu.async_copy(rows.at[b], acc_sh.at[dbuf.at[b]],
                                 ssem.at[b], add=True)

            @pl.when(i + 2 < NCHUNK)
            def _():
                drain_idx(i + 2, rn)
                issue_gather(rn)
        return carry

    lax.fori_loop(0, NOUT, outer, 0)
    plsc.subcore_barrier()
    for q in range(QF):
        pltpu.sync_copy(acc_sh.at[pl.ds(s * RPT + q * K, K)], rows.at[0])
        pltpu.sync_copy(rows.at[0], out_hbm.at[c, pl.ds(s * RPT + q * K, K), :])
    pltpu.sync_copy(acc_sh.at[pl.ds(s * RPT + QF * K, QT)],
                    rows.at[0, pl.ds(0, QT)])
    pltpu.sync_copy(rows.at[0, pl.ds(0, QT)],
                    out_hbm.at[c, pl.ds(s * RPT + QF * K, QT), :])

    @pl.when(s == NS - 1)
    def _():
        pltpu.sync_copy(acc_sh.at[pl.ds(NS * RPT, TAIL)],
                        rows.at[1, pl.ds(0, TAIL)])
        pltpu.sync_copy(rows.at[1, pl.ds(0, TAIL)],
                        out_hbm.at[c, pl.ds(NS * RPT, TAIL), :])


def _dot(a, b):
    return lax.dot_general(a, b, (((1,), (0,)), ((), ())),
                           preferred_element_type=jnp.float32)


def _leaky(x):
    return jnp.where(x > 0, x, 0.01 * x)


def _bn(z, g, b):
    mu = jnp.mean(z, axis=0)
    var = jnp.mean((z - mu) ** 2, axis=0)
    return (z - mu) * lax.rsqrt(var + 1e-5) * g + b


def _tc1_body(x_ref, w1_ref, degp_ref, dinv_ref, hsh_ref):
    deg = degp_ref[pl.ds(0, N)] + degp_ref[pl.ds(N, N)] + 1.0
    dinv = jnp.where(deg > 0, lax.rsqrt(deg), 0.0)
    dinv_ref[...] = dinv
    h = _dot(x_ref[...], w1_ref[...])
    hsh_ref[...] = h * dinv[:, None]


_tc1 = pl.pallas_call(
    _tc1_body,
    out_shape=(jax.ShapeDtypeStruct((N,), jnp.float32),
               jax.ShapeDtypeStruct((N, C_HID), jnp.float32)),
)


def _tc2_body(s_ref, hsh_in_ref, dinv_ref, b1_ref, g1_ref, be1_ref, w2_ref,
              hsh_ref, h_ref):
    dinv = dinv_ref[...]
    z = (s_ref[0] + s_ref[1] + hsh_in_ref[...]) * dinv[:, None] + b1_ref[...]
    h = _leaky(_bn(z, g1_ref[...], be1_ref[...]))
    h_ref[...] = h
    h2 = _dot(h, w2_ref[...])
    hsh_ref[...] = h2 * dinv[:, None]


_tc2 = pl.pallas_call(
    _tc2_body,
    out_shape=(jax.ShapeDtypeStruct((N, C_HID), jnp.float32),
               jax.ShapeDtypeStruct((N, C_HID), jnp.float32)),
)


def _tc3_body(s_ref, hsh_in_ref, dinv_ref, b2_ref, g2_ref, be2_ref, h_ref,
              wl1_ref, bl1_ref, wl2_ref, bl2_ref, out_ref):
    dinv = dinv_ref[...]
    z = (s_ref[0] + s_ref[1] + hsh_in_ref[...]) * dinv[:, None] + b2_ref[...]
    t = _leaky(_bn(z, g2_ref[...], be2_ref[...]) + h_ref[...])
    t = _leaky(_dot(t, wl1_ref[...]) + bl1_ref[...])
    out_ref[...] = _dot(t, wl2_ref[...]) + bl2_ref[...]


_tc3 = pl.pallas_call(
    _tc3_body,
    out_shape=jax.ShapeDtypeStruct((N, C_OUT), jnp.float32),
)


def kernel(x, edge_index, edge_attr, W1, b1, g1, be1, W2, b2, g2, be2,
           Wl1, bl1, Wl2, bl2):
    src = edge_index[0]
    dst = edge_index[1]

    degp = _deg_kernel(dst, edge_attr)
    dinv, hsh1 = _tc1(x, W1, degp)
    s1 = _scatter_kernel(hsh1, src, dst, edge_attr)
    hsh2, h1 = _tc2(s1, hsh1, dinv, b1, g1, be1, W2)
    s2 = _scatter_kernel(hsh2, src, dst, edge_attr)
    return _tc3(s2, hsh2, dinv, b2, g2, be2, h1, Wl1, bl1, Wl2, bl2)


# speedup vs baseline: 25.6237x; 1.4240x over previous
import functools

import jax
import jax.numpy as jnp
from jax import lax
from jax.experimental import pallas as pl
from jax.experimental.pallas import tpu as pltpu
from jax.experimental.pallas import tpu_sc as plsc

N = 10000
E = 320000
C_IN = 128
C_HID = 128
C_OUT = 16

NC = 2
NS = 16
NW = NC * NS
EPT = E // NW
KD = 80
NCHD = EPT // KD
K = 80
NCHUNK = EPT // K
NBUF = 4
RPT = 624
TAIL = N - NS * RPT
QF = RPT // K
QT = RPT - QF * K

_mesh = plsc.VectorSubcoreMesh(core_axis_name="c", subcore_axis_name="s")


@functools.partial(
    pl.kernel,
    out_type=jax.ShapeDtypeStruct((NC * N,), jnp.float32),
    mesh=_mesh,
    scratch_types=[
        pltpu.VMEM((NBUF, KD), jnp.int32),
        pltpu.VMEM((NBUF, KD), jnp.float32),
        pltpu.VMEM((RPT,), jnp.float32),
        pltpu.VMEM_SHARED((N,), jnp.float32),
        pltpu.SemaphoreType.DMA((NBUF,)),
    ],
)
def _deg_kernel(dst_hbm, ew_hbm, out_hbm, idx_v, ew_v, zbuf, acc_sh, isem):
    c = lax.axis_index("c")
    s = lax.axis_index("s")
    wid = s * NC + c

    def zfill(i, carry):
        zbuf[pl.ds(i * 16, 16)] = jnp.zeros((16,), jnp.float32)
        return carry

    lax.fori_loop(0, RPT // 16, zfill, 0)
    pltpu.sync_copy(zbuf, acc_sh.at[pl.ds(s * RPT, RPT)])

    @pl.when(s == NS - 1)
    def _():
        pltpu.sync_copy(zbuf.at[pl.ds(0, TAIL)],
                        acc_sh.at[pl.ds(NS * RPT, TAIL)])

    plsc.subcore_barrier()
    base = wid * EPT

    def issue(j, r):
        off = base + j * KD
        pltpu.async_copy(dst_hbm.at[pl.ds(off, KD)], idx_v.at[r], isem.at[r])
        pltpu.async_copy(ew_hbm.at[pl.ds(off, KD)], ew_v.at[r], isem.at[r])

    def drain(j, r):
        off = base + j * KD
        pltpu.make_async_copy(dst_hbm.at[pl.ds(off, KD)], idx_v.at[r],
                              isem.at[r]).wait()
        pltpu.make_async_copy(ew_hbm.at[pl.ds(off, KD)], ew_v.at[r],
                              isem.at[r]).wait()

    issue(0, 0)
    issue(1, 1)
    NOUTD = (NCHD + NBUF - 1) // NBUF

    def body(g, carry):
        for b in range(NBUF):
            i = g * NBUF + b
            rn = (b + 2) % NBUF

            @pl.when(i + 2 < NCHD)
            def _():
                issue(i + 2, rn)

            @pl.when(i < NCHD)
            def _():
                drain(i, b)
                pltpu.sync_copy(ew_v.at[b], acc_sh.at[idx_v.at[b]], add=True)
        return carry

    lax.fori_loop(0, NOUTD, body, 0)
    plsc.subcore_barrier()
    pltpu.sync_copy(acc_sh.at[pl.ds(s * RPT, RPT)], zbuf)
    pltpu.sync_copy(zbuf, out_hbm.at[pl.ds(c * N + s * RPT, RPT)])

    @pl.when(s == NS - 1)
    def _():
        pltpu.sync_copy(acc_sh.at[pl.ds(NS * RPT, TAIL)],
                        zbuf.at[pl.ds(0, TAIL)])
        pltpu.sync_copy(zbuf.at[pl.ds(0, TAIL)],
                        out_hbm.at[pl.ds(c * N + NS * RPT, TAIL)])


@functools.partial(
    pl.kernel,
    out_type=jax.ShapeDtypeStruct((NC, N, C_HID), jnp.float32),
    mesh=_mesh,
    scratch_types=[
        pltpu.VMEM((NBUF, K), jnp.int32),
        pltpu.VMEM((NBUF, K), jnp.int32),
        pltpu.VMEM((NBUF, K + 16), jnp.float32),
        pltpu.VMEM((NBUF, K, C_HID), jnp.float32),
        pltpu.VMEM_SHARED((N, C_HID), jnp.float32),
        pltpu.SemaphoreType.DMA((NBUF,)),
        pltpu.SemaphoreType.DMA((NBUF,)),
        pltpu.SemaphoreType.DMA((NBUF,)),
    ],
)
def _scatter_kernel(hsh_hbm, src_hbm, dst_hbm, ew_hbm, out_hbm,
                    sbuf, dbuf, ebuf, rows, acc_sh, gsem, ssem, isem):
    c = lax.axis_index("c")
    s = lax.axis_index("s")
    wid = s * NC + c
    base = wid * EPT

    def zfill(i, carry):
        rr = i // (C_HID // 16)
        col = (i % (C_HID // 16)) * 16
        rows[0, rr, pl.ds(col, 16)] = jnp.zeros((16,), jnp.float32)
        return carry

    lax.fori_loop(0, K * (C_HID // 16), zfill, 0)
    for q in range(QF):
        pltpu.sync_copy(rows.at[0], acc_sh.at[pl.ds(s * RPT + q * K, K)])
    pltpu.sync_copy(rows.at[0, pl.ds(0, QT)],
                    acc_sh.at[pl.ds(s * RPT + QF * K, QT)])

    @pl.when(s == NS - 1)
    def _():
        pltpu.sync_copy(rows.at[0, pl.ds(0, TAIL)],
                        acc_sh.at[pl.ds(NS * RPT, TAIL)])

    plsc.subcore_barrier()

    def issue_idx(j, r):
        off = base + j * K
        pltpu.async_copy(src_hbm.at[pl.ds(off, K)], sbuf.at[r], isem.at[r])
        pltpu.async_copy(dst_hbm.at[pl.ds(off, K)], dbuf.at[r], isem.at[r])
        pltpu.async_copy(ew_hbm.at[pl.ds(off, K)],
                         ebuf.at[r, pl.ds(0, K)], isem.at[r])

    def drain_idx(j, r):
        off = base + j * K
        pltpu.make_async_copy(src_hbm.at[pl.ds(off, K)], sbuf.at[r],
                              isem.at[r]).wait()
        pltpu.make_async_copy(dst_hbm.at[pl.ds(off, K)], dbuf.at[r],
                              isem.at[r]).wait()
        pltpu.make_async_copy(ew_hbm.at[pl.ds(off, K)],
                              ebuf.at[r, pl.ds(0, K)], isem.at[r]).wait()

    def issue_gather(r):
        pltpu.async_copy(hsh_hbm.at[sbuf.at[r]], rows.at[r], gsem.at[r])

    issue_idx(0, 0)
    issue_idx(1, 1)
    drain_idx(0, 0)
    issue_gather(0)
    drain_idx(1, 1)
    issue_gather(1)

    NOUT = (NCHUNK + NBUF - 1) // NBUF + 1

    def outer(g, carry):
        for b in range(NBUF):
            i = g * NBUF + b
            rn = (b + 2) % NBUF

            @pl.when(jnp.logical_and(i >= 2, i - 2 < NCHUNK))
            def _():
                pltpu.make_async_copy(rows.at[rn], acc_sh.at[dbuf.at[rn]],
                                      ssem.at[rn]).wait()

            @pl.when(i + 2 < NCHUNK)
            def _():
                issue_idx(i + 2, rn)

            @pl.when(i < NCHUNK)
            def _():
                pltpu.make_async_copy(hsh_hbm.at[sbuf.at[b]], rows.at[b],
                                      gsem.at[b]).wait()

                def scale(k, carry2):
                    v = ebuf[b, pl.ds(k, 16)]
                    bb = jnp.zeros((16,), jnp.float32) + v[0]
                    for jj in range(C_HID // 16):
                        rows[b, k, pl.ds(jj * 16, 16)] = (
                            rows[b, k, pl.ds(jj * 16, 16)] * bb)
                    return carry2

                lax.fori_loop(0, K, scale, 0, unroll=4)
                pltpu.async_copy(rows.at[b], acc_sh.at[dbuf.at[b]],
                                 ssem.at[b], add=True)

            @pl.when(i + 2 < NCHUNK)
            def _():
                drain_idx(i + 2, rn)
                issue_gather(rn)
        return carry

    lax.fori_loop(0, NOUT, outer, 0)
    plsc.subcore_barrier()
    for q in range(QF):
        pltpu.sync_copy(acc_sh.at[pl.ds(s * RPT + q * K, K)], rows.at[0])
        pltpu.sync_copy(rows.at[0], out_hbm.at[c, pl.ds(s * RPT + q * K, K), :])
    pltpu.sync_copy(acc_sh.at[pl.ds(s * RPT + QF * K, QT)],
                    rows.at[0, pl.ds(0, QT)])
    pltpu.sync_copy(rows.at[0, pl.ds(0, QT)],
                    out_hbm.at[c, pl.ds(s * RPT + QF * K, QT), :])

    @pl.when(s == NS - 1)
    def _():
        pltpu.sync_copy(acc_sh.at[pl.ds(NS * RPT, TAIL)],
                        rows.at[1, pl.ds(0, TAIL)])
        pltpu.sync_copy(rows.at[1, pl.ds(0, TAIL)],
                        out_hbm.at[c, pl.ds(NS * RPT, TAIL), :])


def _dot(a, b):
    return lax.dot_general(a, b, (((1,), (0,)), ((), ())),
                           preferred_element_type=jnp.float32)


def _leaky(x):
    return jnp.where(x > 0, x, 0.01 * x)


def _bn(z, g, b):
    mu = jnp.mean(z, axis=0)
    var = jnp.mean((z - mu) ** 2, axis=0)
    return (z - mu) * lax.rsqrt(var + 1e-5) * g + b


def _tc1_body(x_ref, w1_ref, degp_ref, dinv_ref, hsh_ref):
    deg = degp_ref[pl.ds(0, N)] + degp_ref[pl.ds(N, N)] + 1.0
    dinv = jnp.where(deg > 0, lax.rsqrt(deg), 0.0)
    dinv_ref[...] = dinv
    h = _dot(x_ref[...], w1_ref[...])
    hsh_ref[...] = h * dinv[:, None]


_tc1 = pl.pallas_call(
    _tc1_body,
    out_shape=(jax.ShapeDtypeStruct((N,), jnp.float32),
               jax.ShapeDtypeStruct((N, C_HID), jnp.float32)),
)


def _tc2_body(s_ref, hsh_in_ref, dinv_ref, b1_ref, g1_ref, be1_ref, w2_ref,
              hsh_ref, h_ref):
    dinv = dinv_ref[...]
    z = (s_ref[0] + s_ref[1] + hsh_in_ref[...]) * dinv[:, None] + b1_ref[...]
    h = _leaky(_bn(z, g1_ref[...], be1_ref[...]))
    h_ref[...] = h
    h2 = _dot(h, w2_ref[...])
    hsh_ref[...] = h2 * dinv[:, None]


_tc2 = pl.pallas_call(
    _tc2_body,
    out_shape=(jax.ShapeDtypeStruct((N, C_HID), jnp.float32),
               jax.ShapeDtypeStruct((N, C_HID), jnp.float32)),
)


def _tc3_body(s_ref, hsh_in_ref, dinv_ref, b2_ref, g2_ref, be2_ref, h_ref,
              wl1_ref, bl1_ref, wl2_ref, bl2_ref, out_ref):
    dinv = dinv_ref[...]
    z = (s_ref[0] + s_ref[1] + hsh_in_ref[...]) * dinv[:, None] + b2_ref[...]
    t = _leaky(_bn(z, g2_ref[...], be2_ref[...]) + h_ref[...])
    t = _leaky(_dot(t, wl1_ref[...]) + bl1_ref[...])
    out_ref[...] = _dot(t, wl2_ref[...]) + bl2_ref[...]


_tc3 = pl.pallas_call(
    _tc3_body,
    out_shape=jax.ShapeDtypeStruct((N, C_OUT), jnp.float32),
)


def kernel(x, edge_index, edge_attr, W1, b1, g1, be1, W2, b2, g2, be2,
           Wl1, bl1, Wl2, bl2):
    src = edge_index[0]
    dst = edge_index[1]

    degp = _deg_kernel(dst, edge_attr)
    dinv, hsh1 = _tc1(x, W1, degp)
    s1 = _scatter_kernel(hsh1, src, dst, edge_attr)
    hsh2, h1 = _tc2(s1, hsh1, dinv, b1, g1, be1, W2)
    s2 = _scatter_kernel(hsh2, src, dst, edge_attr)
    return _tc3(s2, hsh2, dinv, b2, g2, be2, h1, Wl1, bl1, Wl2, bl2)
